# baseline (speedup 1.0000x reference)
import functools

import jax
import jax.numpy as jnp
from jax import lax
from jax.experimental import pallas as pl
from jax.experimental.pallas import tpu as pltpu
from jax.experimental.pallas import tpu_sc as plsc

L = 16
NC = 2
NS = 16
NW = NC * NS
B = 16384
D = 64
BPW = B // NW
CH = 32
NCH2 = BPW // CH
TB = 8
NBLK = 125000


def _dot_kernel(user_ids, item_ids, uf3, if3, ib3):
    mesh = plsc.VectorSubcoreMesh(core_axis_name="c", subcore_axis_name="s")

    @functools.partial(
        pl.kernel,
        out_type=jax.ShapeDtypeStruct((B,), jnp.float32),
        mesh=mesh,
        compiler_params=pltpu.CompilerParams(needs_layout_passes=False),
        scratch_types=[
            pltpu.VMEM((NCH2, CH), jnp.int32),
            pltpu.VMEM((NCH2, CH), jnp.int32),
            pltpu.VMEM((CH, TB, D), jnp.float32),
            pltpu.VMEM((CH, TB, D), jnp.float32),
            pltpu.VMEM((CH, TB, 1), jnp.float32),
            pltpu.VMEM((BPW,), jnp.float32),
            pltpu.SemaphoreType.DMA,
        ],
    )
    def run(uids_hbm, iids_hbm, uf_hbm, if_hbm, ib_hbm, out_hbm,
            uidx, iidx, ublocks, iblocks, bblocks, outv, sem):
        wid = lax.axis_index("s") * NC + lax.axis_index("c")
        base = wid * BPW

        for c in range(NCH2):
            pltpu.sync_copy(uids_hbm.at[pl.ds(base + c * CH, CH)],
                            uidx.at[c])
            pltpu.sync_copy(iids_hbm.at[pl.ds(base + c * CH, CH)],
                            iidx.at[c])

        def chunk_body(c, carry):
            for g in range(CH // L):
                uvec = jax.lax.shift_right_logical(uidx[c, pl.ds(g * L, L)], 3)
                ivec = jax.lax.shift_right_logical(iidx[c, pl.ds(g * L, L)], 3)
                for j in range(L):
                    r = g * L + j
                    pltpu.make_async_copy(
                        uf_hbm.at[uvec[j]], ublocks.at[r], sem).start()
                    pltpu.make_async_copy(
                        if_hbm.at[ivec[j]], iblocks.at[r], sem).start()
                    pltpu.make_async_copy(
                        ib_hbm.at[ivec[j]], bblocks.at[r], sem).start()
            pltpu.make_async_copy(
                uf_hbm.at[pl.ds(0, CH)], ublocks, sem).wait()
            pltpu.make_async_copy(
                if_hbm.at[pl.ds(0, CH)], iblocks, sem).wait()
            pltpu.make_async_copy(
                ib_hbm.at[pl.ds(0, CH)], bblocks, sem).wait()

            zeros = jnp.zeros((L,), jnp.int32)
            for g in range(CH // L):
                sl = pl.ds(g * L, L)
                jvec = lax.iota(jnp.int32, L) + g * L
                urow = jnp.bitwise_and(uidx[c, sl], 7)
                irow = jnp.bitwise_and(iidx[c, sl], 7)
                acc = plsc.load_gather(bblocks, [jvec, irow, zeros])
                for d in range(D):
                    col = jnp.full((L,), d, jnp.int32)
                    u = plsc.load_gather(ublocks, [jvec, urow, col])
                    it = plsc.load_gather(iblocks, [jvec, irow, col])
                    acc = acc + u * it
                outv[pl.ds(c * CH + g * L, L)] = acc
            return carry

        lax.fori_loop(0, NCH2, chunk_body, 0)
        pltpu.sync_copy(outv, out_hbm.at[pl.ds(base, BPW)])

    return run(user_ids, item_ids, uf3, if3, ib3)


def kernel(user_ids, item_ids, user_factors, item_factors, item_bias):
    uf3 = user_factors.reshape(NBLK, TB, D)
    if3 = item_factors.reshape(NBLK, TB, D)
    ib3 = item_bias.reshape(NBLK, TB, 1)
    return _dot_kernel(user_ids, item_ids, uf3, if3, ib3)
